# baseline (speedup 1.0000x reference)
import functools

import jax
import jax.numpy as jnp
from jax import lax
from jax.experimental import pallas as pl
from jax.experimental.pallas import tpu as pltpu
from jax.experimental.pallas import tpu_sc as plsc

R = 512
C = 32
B = 1048576
NC = 2
NS = 16
NW = NC * NS
PPW = B // NW
N = 128
NCH = PPW // N
PAIRS = ((0, 1), (0, 2), (1, 2))
NG = N // 16
NA = 3 * R * (R // 2)


def _grid_body(table_hbm, inp_hbm, out_hbm,
               in_v0, in_v1, idx_v0, idx_v1, w_v0, w_v1,
               rows_v0, rows_v1, out_v0, out_v1,
               sem_g0, sem_g1, sem_o0, sem_o1, sem_i0, sem_i1):
    wid = lax.axis_index("s") * NC + lax.axis_index("c")
    wbase = wid * PPW
    lanes = lax.iota(jnp.int32, 16)

    slots = ((in_v0, idx_v0, w_v0, rows_v0, out_v0, sem_g0, sem_o0),
             (in_v1, idx_v1, w_v1, rows_v1, out_v1, sem_g1, sem_o1))
    in_sems = (sem_i0, sem_i1)

    def fire_in(k, s):
        in_v = slots[s][0]
        sem_i = in_sems[s]
        base = wbase + k * N
        for a in range(3):
            pltpu.async_copy(inp_hbm.at[pl.ds(a * B + base, N)],
                             in_v.at[pl.ds(a * N, N)], sem_i)

    def wait_in(k, s):
        in_v = slots[s][0]
        sem_i = in_sems[s]
        base = wbase + k * N
        for a in range(3):
            pltpu.make_async_copy(inp_hbm.at[pl.ds(a * B + base, N)],
                                  in_v.at[pl.ds(a * N, N)], sem_i).wait()

    def prep_fire(k, s):
        in_v, idx_v, w_v, rows_v, _, sem_g, _ = slots[s]

        def prep_g(g, carry):
            off = g * 16
            coords = []
            for a in range(3):
                x = in_v[pl.ds(a * N + off, 16)]
                t = jnp.clip((x + 1.0) * 0.5, 0.0, 1.0) * float(R - 1)
                ti = jnp.minimum(t.astype(jnp.int32), R - 2)
                tf = t - ti.astype(jnp.float32)
                coords.append((ti, tf))
            for l, (a, b) in enumerate(PAIRS):
                ua, fa = coords[a]
                vb, fb = coords[b]
                odd = jnp.bitwise_and(vb, 1)
                t = jnp.right_shift(vb, 1)
                line0 = l * R + ua
                line1 = line0 + 1
                r0 = jnp.where(odd == 1, NA + line0 * (R // 2 - 1) + t,
                               line0 * (R // 2) + t)
                r1 = jnp.where(odd == 1, NA + line1 * (R // 2 - 1) + t,
                               line1 * (R // 2) + t)
                ga = 1.0 - fa
                gb = 1.0 - fb
                idx_v[2 * l + 0, pl.ds(off, 16)] = r0
                idx_v[2 * l + 1, pl.ds(off, 16)] = r1
                w_v[4 * l + 0, pl.ds(off, 16)] = ga * gb
                w_v[4 * l + 1, pl.ds(off, 16)] = ga * fb
                w_v[4 * l + 2, pl.ds(off, 16)] = fa * gb
                w_v[4 * l + 3, pl.ds(off, 16)] = fa * fb
            return carry

        lax.fori_loop(0, NG, prep_g, 0)
        for j in range(6):
            pltpu.async_copy(table_hbm.at[idx_v.at[j]],
                             rows_v.at[pl.ds(j * N, N)], sem_g)

    def drain_gathers(s):
        _, idx_v, _, rows_v, _, sem_g, _ = slots[s]
        for j in range(6):
            pltpu.make_async_copy(table_hbm.at[idx_v.at[j]],
                                  rows_v.at[pl.ds(j * N, N)], sem_g).wait()

    def interp(k, s):
        _, _, w_v, rows_v, out_v, _, _ = slots[s]

        def g_body(g, carry):
            base16 = g * 16
            w16s = [w_v[j, pl.ds(base16, 16)] for j in range(12)]

            def i_body(i, carry2):
                p = base16 + i
                il = lax.broadcast(i, (16,))
                acc0 = jnp.zeros((16,), jnp.float32)
                acc1 = jnp.zeros((16,), jnp.float32)
                for j in range(6):
                    wa = jnp.take(w16s[2 * j], il)
                    wb = jnp.take(w16s[2 * j + 1], il)
                    ra = rows_v[j * N + p, pl.ds(0, 32)]
                    rb = rows_v[j * N + p, pl.ds(32, 32)]
                    ea0, ea1 = plsc.unpack(ra,
                                           format=plsc.PackFormat.INTERLEAVED)
                    eb0, eb1 = plsc.unpack(rb,
                                           format=plsc.PackFormat.INTERLEAVED)
                    acc0 = acc0 + wa * ea0 + wb * eb0
                    acc1 = acc1 + wa * ea1 + wb * eb1
                out_v[p, pl.ds(0, 16)] = acc0
                out_v[p, pl.ds(16, 16)] = acc1
                return carry2

            lax.fori_loop(0, 16, i_body, 0)
            return carry

        lax.fori_loop(0, NG, g_body, 0)

    def fire_out(k, s):
        out_v, sem_o = slots[s][4], slots[s][6]
        base = wbase + k * N
        pltpu.async_copy(out_v, out_hbm.at[pl.ds(base, N)], sem_o)

    def wait_out(k, s):
        out_v, sem_o = slots[s][4], slots[s][6]
        base = wbase + (k - 2) * N
        pltpu.make_async_copy(out_v, out_hbm.at[pl.ds(base, N)], sem_o).wait()

    def step(k, s):
        nxt = k + 1

        @pl.when(nxt < NCH)
        def _():
            wait_in(nxt, 1 - s)
            prep_fire(nxt, 1 - s)

        @pl.when(k + 2 < NCH)
        def _():
            fire_in(k + 2, s)

        drain_gathers(s)

        @pl.when(k >= 2)
        def _():
            wait_out(k, s)

        interp(k, s)
        fire_out(k, s)

    fire_in(0, 0)
    fire_in(1, 1)
    wait_in(0, 0)
    prep_fire(0, 0)

    def body2(k2, carry):
        k = 2 * k2
        step(k, 0)
        step(k + 1, 1)
        return carry

    lax.fori_loop(0, NCH // 2, body2, 0)
    wait_out(NCH, 0)
    wait_out(NCH + 1, 1)


@jax.jit
def _grid_encode_sc(table, inp_t):
    mesh = plsc.VectorSubcoreMesh(core_axis_name="c", subcore_axis_name="s")
    f = pl.kernel(
        _grid_body,
        out_type=jax.ShapeDtypeStruct((B, C), jnp.float32),
        mesh=mesh,
        compiler_params=pltpu.CompilerParams(needs_layout_passes=False,
                                             use_tc_tiling_on_sc=False),
        scratch_types=[
            pltpu.VMEM((N * 3,), jnp.float32),
            pltpu.VMEM((N * 3,), jnp.float32),
            pltpu.VMEM((6, N), jnp.int32),
            pltpu.VMEM((6, N), jnp.int32),
            pltpu.VMEM((12, N), jnp.float32),
            pltpu.VMEM((12, N), jnp.float32),
            pltpu.VMEM((6 * N, 2 * C), jnp.bfloat16),
            pltpu.VMEM((6 * N, 2 * C), jnp.bfloat16),
            pltpu.VMEM((N, C), jnp.float32),
            pltpu.VMEM((N, C), jnp.float32),
            pltpu.SemaphoreType.DMA,
            pltpu.SemaphoreType.DMA,
            pltpu.SemaphoreType.DMA,
            pltpu.SemaphoreType.DMA,
            pltpu.SemaphoreType.DMA,
            pltpu.SemaphoreType.DMA,
        ],
    )
    return f(table, inp_t)


_COL_PERM = tuple(
    c for i in range(C // 2) for c in (i, C // 2 + i)
)


def kernel(triplane_, inputs):
    tp = jnp.squeeze(triplane_, axis=1)
    tpq = tp[:, jnp.array(_COL_PERM, jnp.int32), :, :]
    tabp = jnp.transpose(tpq, (0, 2, 3, 1)).astype(jnp.bfloat16)
    t_even = tabp.reshape(3 * R * (R // 2), 2 * C)
    t_odd = tabp[:, :, 1:R - 1, :].reshape(3 * R * (R // 2 - 1), 2 * C)
    table = jnp.concatenate([t_even, t_odd], axis=0)
    inp_t = jnp.transpose(inputs).reshape(3 * B)
    return _grid_encode_sc(table, inp_t)


# speedup vs baseline: 1.5635x; 1.5635x over previous
import functools

import jax
import jax.numpy as jnp
from jax import lax
from jax.experimental import pallas as pl
from jax.experimental.pallas import tpu as pltpu
from jax.experimental.pallas import tpu_sc as plsc

R = 512
C = 32
B = 1048576
NC = 2
NS = 16
NW = NC * NS
PPW = B // NW
N = 128
NCH = PPW // N
PAIRS = ((0, 1), (0, 2), (1, 2))
NG = N // 16


def _grid_body(table_hbm, inp_hbm, out_hbm,
               in_v0, in_v1, idx_v0, idx_v1, w_v0, w_v1,
               rows_v0, rows_v1, out_v0, out_v1,
               sem_g0, sem_g1, sem_o0, sem_o1, sem_i0, sem_i1):
    wid = lax.axis_index("s") * NC + lax.axis_index("c")
    wbase = wid * PPW
    lanes = lax.iota(jnp.int32, 16)

    slots = ((in_v0, idx_v0, w_v0, rows_v0, out_v0, sem_g0, sem_o0),
             (in_v1, idx_v1, w_v1, rows_v1, out_v1, sem_g1, sem_o1))
    in_sems = (sem_i0, sem_i1)

    def fire_in(k, s):
        in_v = slots[s][0]
        sem_i = in_sems[s]
        base = wbase + k * N
        for a in range(3):
            pltpu.async_copy(inp_hbm.at[pl.ds(a * B + base, N)],
                             in_v.at[pl.ds(a * N, N)], sem_i)

    def wait_in(k, s):
        in_v = slots[s][0]
        sem_i = in_sems[s]
        base = wbase + k * N
        for a in range(3):
            pltpu.make_async_copy(inp_hbm.at[pl.ds(a * B + base, N)],
                                  in_v.at[pl.ds(a * N, N)], sem_i).wait()

    def prep_fire(k, s):
        in_v, idx_v, w_v, rows_v, _, sem_g, _ = slots[s]

        def prep_g(g, carry):
            off = g * 16
            coords = []
            for a in range(3):
                x = in_v[pl.ds(a * N + off, 16)]
                t = jnp.clip((x + 1.0) * 0.5, 0.0, 1.0) * float(R - 1)
                ti = jnp.minimum(t.astype(jnp.int32), R - 2)
                tf = t - ti.astype(jnp.float32)
                coords.append((ti, tf))
            for l, (a, b) in enumerate(PAIRS):
                ua, fa = coords[a]
                vb, fb = coords[b]
                i00 = ua * R + vb + (l * R * R)
                ga = 1.0 - fa
                gb = 1.0 - fb
                idx_v[4 * l + 0, pl.ds(off, 16)] = i00
                idx_v[4 * l + 1, pl.ds(off, 16)] = i00 + 1
                idx_v[4 * l + 2, pl.ds(off, 16)] = i00 + R
                idx_v[4 * l + 3, pl.ds(off, 16)] = i00 + (R + 1)
                w_v[4 * l + 0, pl.ds(off, 16)] = ga * gb
                w_v[4 * l + 1, pl.ds(off, 16)] = ga * fb
                w_v[4 * l + 2, pl.ds(off, 16)] = fa * gb
                w_v[4 * l + 3, pl.ds(off, 16)] = fa * fb
            return carry

        lax.fori_loop(0, NG, prep_g, 0)
        for j in range(12):
            pltpu.async_copy(table_hbm.at[idx_v.at[j]],
                             rows_v.at[pl.ds(j * N, N)], sem_g)

    def drain_gathers(s):
        _, idx_v, _, rows_v, _, sem_g, _ = slots[s]
        for j in range(12):
            pltpu.make_async_copy(table_hbm.at[idx_v.at[j]],
                                  rows_v.at[pl.ds(j * N, N)], sem_g).wait()

    def interp(k, s):
        _, _, w_v, rows_v, out_v, _, _ = slots[s]

        def g_body(g, carry):
            base16 = g * 16
            w16s = [w_v[j, pl.ds(base16, 16)] for j in range(12)]

            def i_body(i, carry2):
                p = base16 + i
                il = lax.broadcast(i, (16,))
                acc0 = jnp.zeros((16,), jnp.float32)
                acc1 = jnp.zeros((16,), jnp.float32)
                for j in range(12):
                    wj = jnp.take(w16s[j], il)
                    r0 = rows_v[j * N + p, pl.ds(0, 16)]
                    r1 = rows_v[j * N + p, pl.ds(16, 16)]
                    acc0 = acc0 + wj * r0
                    acc1 = acc1 + wj * r1
                out_v[p, pl.ds(0, 16)] = acc0
                out_v[p, pl.ds(16, 16)] = acc1
                return carry2

            lax.fori_loop(0, 16, i_body, 0)
            return carry

        lax.fori_loop(0, NG, g_body, 0)

    def fire_out(k, s):
        out_v, sem_o = slots[s][4], slots[s][6]
        base = wbase + k * N
        pltpu.async_copy(out_v, out_hbm.at[pl.ds(base, N)], sem_o)

    def wait_out(k, s):
        out_v, sem_o = slots[s][4], slots[s][6]
        base = wbase + (k - 2) * N
        pltpu.make_async_copy(out_v, out_hbm.at[pl.ds(base, N)], sem_o).wait()

    def step(k, s):
        nxt = k + 1

        @pl.when(nxt < NCH)
        def _():
            wait_in(nxt, 1 - s)
            prep_fire(nxt, 1 - s)

        @pl.when(k + 2 < NCH)
        def _():
            fire_in(k + 2, s)

        drain_gathers(s)

        @pl.when(k >= 2)
        def _():
            wait_out(k, s)

        interp(k, s)
        fire_out(k, s)

    fire_in(0, 0)
    fire_in(1, 1)
    wait_in(0, 0)
    prep_fire(0, 0)

    def body2(k2, carry):
        k = 2 * k2
        step(k, 0)
        step(k + 1, 1)
        return carry

    lax.fori_loop(0, NCH // 2, body2, 0)
    wait_out(NCH, 0)
    wait_out(NCH + 1, 1)


@jax.jit
def _grid_encode_sc(table, inp_t):
    mesh = plsc.VectorSubcoreMesh(core_axis_name="c", subcore_axis_name="s")
    f = pl.kernel(
        _grid_body,
        out_type=jax.ShapeDtypeStruct((B, C), jnp.float32),
        mesh=mesh,
        compiler_params=pltpu.CompilerParams(needs_layout_passes=False,
                                             use_tc_tiling_on_sc=False),
        scratch_types=[
            pltpu.VMEM((N * 3,), jnp.float32),
            pltpu.VMEM((N * 3,), jnp.float32),
            pltpu.VMEM((12, N), jnp.int32),
            pltpu.VMEM((12, N), jnp.int32),
            pltpu.VMEM((12, N), jnp.float32),
            pltpu.VMEM((12, N), jnp.float32),
            pltpu.VMEM((12 * N, C), jnp.float32),
            pltpu.VMEM((12 * N, C), jnp.float32),
            pltpu.VMEM((N, C), jnp.float32),
            pltpu.VMEM((N, C), jnp.float32),
            pltpu.SemaphoreType.DMA,
            pltpu.SemaphoreType.DMA,
            pltpu.SemaphoreType.DMA,
            pltpu.SemaphoreType.DMA,
            pltpu.SemaphoreType.DMA,
            pltpu.SemaphoreType.DMA,
        ],
    )
    return f(table, inp_t)


def kernel(triplane_, inputs):
    tp = jnp.squeeze(triplane_, axis=1)
    table = jnp.transpose(tp, (0, 2, 3, 1)).reshape(3 * R * R, C)
    inp_t = jnp.transpose(inputs).reshape(3 * B)
    return _grid_encode_sc(table, inp_t)
